# baseline (speedup 1.0000x reference)
import functools

import jax
import jax.numpy as jnp
from jax import lax
from jax.experimental import pallas as pl
from jax.experimental.pallas import tpu as pltpu
from jax.experimental.pallas import tpu_sc as plsc

_EMBED = 256
_IMG = 224
_NC = 2
_NS = 16
_NW = _NC * _NS
_L = 16
_CHUNK = 64
_NBUF = 3
_LOOK = 2
_NSLICE = 8
_TCBLK = 2048


def _sc_gather_body(n_tok, fx_hbm, fy_hbm, pos_hbm, rows_hbm,
                    fx_v, fy_v, idx_v, rows_v, sem_g, sem_o):
  b_per_w = n_tok // _NW
  n_chunks = b_per_w // _CHUNK
  wid = lax.axis_index("s") * _NC + lax.axis_index("c")
  w_base = wid * b_per_w

  pltpu.sync_copy(fx_hbm.at[pl.ds(w_base, b_per_w)], fx_v)
  pltpu.sync_copy(fy_hbm.at[pl.ds(w_base, b_per_w)], fy_v)

  def compute_idx(i, carry):
    sl = pl.ds(i * _L, _L)
    xi = jnp.clip((fx_v[sl] * _IMG).astype(jnp.int32), 0, _IMG - 1)
    yi = jnp.clip((fy_v[sl] * _IMG).astype(jnp.int32), 0, _IMG - 1)
    idx_v[sl] = yi * _IMG + xi
    return carry

  lax.fori_loop(0, b_per_w // _L, compute_idx, 0)

  def start_g(c, b):
    pltpu.async_copy(pos_hbm.at[idx_v.at[pl.ds(c * _CHUNK, _CHUNK)]],
                     rows_v.at[b], sem_g.at[b])

  def wait_g(b):
    pltpu.make_async_copy(rows_hbm.at[pl.ds(0, _CHUNK)],
                          rows_v.at[b], sem_g.at[b]).wait()

  def start_s(c, b):
    pltpu.async_copy(rows_v.at[b],
                     rows_hbm.at[pl.ds(w_base + c * _CHUNK, _CHUNK)],
                     sem_o.at[b])

  def wait_s(b):
    pltpu.make_async_copy(rows_v.at[b],
                          rows_hbm.at[pl.ds(0, _CHUNK)], sem_o.at[b]).wait()

  for j in range(_LOOK):
    start_g(j, j)

  def chunk_step(c, carry):
    b = lax.rem(c, _NBUF)
    bn = lax.rem(c + _LOOK, _NBUF)

    @pl.when(c + _LOOK < n_chunks)
    def _():
      @pl.when(c + _LOOK >= _NBUF)
      def _():
        wait_s(bn)
      start_g(c + _LOOK, bn)

    wait_g(b)
    start_s(c, b)
    return carry

  lax.fori_loop(0, n_chunks, chunk_step, 0)

  for b in range(_NBUF):
    wait_s(b)


def _sc_gather(fx, fy, pos):
  n_tok = fx.shape[0]
  b_per_w = n_tok // _NW
  mesh = plsc.VectorSubcoreMesh(core_axis_name="c", subcore_axis_name="s")
  return pl.kernel(
      functools.partial(_sc_gather_body, n_tok),
      out_type=jax.ShapeDtypeStruct((n_tok, _EMBED), jnp.float32),
      mesh=mesh,
      scratch_types=[
          pltpu.VMEM((b_per_w,), jnp.float32),
          pltpu.VMEM((b_per_w,), jnp.float32),
          pltpu.VMEM((b_per_w,), jnp.int32),
          pltpu.VMEM((_NBUF, _CHUNK, _EMBED), jnp.float32),
          pltpu.SemaphoreType.DMA((_NBUF,)),
          pltpu.SemaphoreType.DMA((_NBUF,)),
      ],
  )(fx, fy, pos)


def _tc_add_body(a_ref, b_ref, o_ref):
  o_ref[...] = a_ref[...] + b_ref[...]


def _tc_add(desc, rows):
  n = desc.shape[0]
  return pl.pallas_call(
      _tc_add_body,
      grid=(n // _TCBLK,),
      in_specs=[pl.BlockSpec((_TCBLK, _EMBED), lambda i: (i, 0)),
                pl.BlockSpec((_TCBLK, _EMBED), lambda i: (i, 0))],
      out_specs=pl.BlockSpec((_TCBLK, _EMBED), lambda i: (i, 0)),
      out_shape=jax.ShapeDtypeStruct((n, _EMBED), jnp.float32),
  )(desc, rows)


@jax.jit
def kernel(discriptors, flows_in, pos_2d):
  shape = discriptors.shape
  n_tok = shape[0] * shape[1]
  d = discriptors.reshape(n_tok, _EMBED)
  fx = flows_in[..., 0].reshape(n_tok)
  fy = flows_in[..., 1].reshape(n_tok)
  p = pos_2d.reshape(_IMG * _IMG, _EMBED)

  sl_tok = n_tok // _NSLICE
  rows = [_sc_gather(fx[s * sl_tok:(s + 1) * sl_tok],
                     fy[s * sl_tok:(s + 1) * sl_tok], p)
          for s in range(_NSLICE)]
  outs = [_tc_add(d[s * sl_tok:(s + 1) * sl_tok], rows[s])
          for s in range(_NSLICE)]
  return jnp.concatenate(outs, axis=0).reshape(shape)


# speedup vs baseline: 2.7148x; 2.7148x over previous
import functools

import jax
import jax.numpy as jnp
from jax import lax
from jax.experimental import pallas as pl
from jax.experimental.pallas import tpu as pltpu
from jax.experimental.pallas import tpu_sc as plsc

_EMBED = 256
_IMG = 224
_NC = 2
_NS = 16
_NW = _NC * _NS
_L = 16
_CHUNK = 64
_NBUF = 3


def _sc_body(n_tok, fx_hbm, fy_hbm, desc_hbm, pos_hbm, out_hbm,
             fx_v, fy_v, idx_v, rows_v, desc_v, sem_g, sem_d, sem_o):
  b_per_w = n_tok // _NW
  n_chunks = b_per_w // _CHUNK
  wid = lax.axis_index("s") * _NC + lax.axis_index("c")
  w_base = wid * b_per_w

  pltpu.sync_copy(fx_hbm.at[pl.ds(w_base, b_per_w)], fx_v)
  pltpu.sync_copy(fy_hbm.at[pl.ds(w_base, b_per_w)], fy_v)

  def compute_idx(i, carry):
    sl = pl.ds(i * _L, _L)
    xi = jnp.clip((fx_v[sl] * _IMG).astype(jnp.int32), 0, _IMG - 1)
    yi = jnp.clip((fy_v[sl] * _IMG).astype(jnp.int32), 0, _IMG - 1)
    idx_v[sl] = yi * _IMG + xi
    return carry

  lax.fori_loop(0, b_per_w // _L, compute_idx, 0)

  def start_in(c, b):
    pltpu.async_copy(pos_hbm.at[idx_v.at[pl.ds(c * _CHUNK, _CHUNK)]],
                     rows_v.at[b], sem_g.at[b])
    pltpu.async_copy(desc_hbm.at[pl.ds(w_base + c * _CHUNK, _CHUNK)],
                     desc_v.at[b], sem_d.at[b])

  def wait_in(b):
    pltpu.make_async_copy(desc_hbm.at[pl.ds(0, _CHUNK)],
                          rows_v.at[b], sem_g.at[b]).wait()
    pltpu.make_async_copy(desc_hbm.at[pl.ds(0, _CHUNK)],
                          desc_v.at[b], sem_d.at[b]).wait()

  def start_s(c, b):
    pltpu.async_copy(desc_v.at[b],
                     out_hbm.at[pl.ds(w_base + c * _CHUNK, _CHUNK)],
                     sem_o.at[b])

  def wait_s(b):
    pltpu.make_async_copy(desc_v.at[b],
                          out_hbm.at[pl.ds(0, _CHUNK)], sem_o.at[b]).wait()

  start_in(0, 0)
  start_in(1, 1)

  def chunk_step(c, carry):
    b = lax.rem(c, _NBUF)
    bn = lax.rem(c + 2, _NBUF)

    @pl.when(c + 2 < n_chunks)
    def _():
      @pl.when(c + 2 >= _NBUF)
      def _():
        wait_s(bn)
      start_in(c + 2, bn)

    wait_in(b)

    def add_row(r2, carry2):
      for u in range(2):
        r = r2 * 2 + u
        for k in range(_EMBED // _L):
          sl = pl.ds(k * _L, _L)
          plsc.addupdate(desc_v.at[b, r, sl], rows_v[b, r, sl])
      return carry2

    lax.fori_loop(0, _CHUNK // 2, add_row, 0)
    start_s(c, b)
    return carry

  lax.fori_loop(0, n_chunks, chunk_step, 0)

  for b in range(_NBUF):
    wait_s(b)


@jax.jit
def kernel(discriptors, flows_in, pos_2d):
  shape = discriptors.shape
  n_tok = shape[0] * shape[1]
  d = discriptors.reshape(n_tok, _EMBED)
  fx = flows_in[..., 0].reshape(n_tok)
  fy = flows_in[..., 1].reshape(n_tok)
  p = pos_2d.reshape(_IMG * _IMG, _EMBED)

  b_per_w = n_tok // _NW
  mesh = plsc.VectorSubcoreMesh(core_axis_name="c", subcore_axis_name="s")
  out = pl.kernel(
      functools.partial(_sc_body, n_tok),
      out_type=jax.ShapeDtypeStruct((n_tok, _EMBED), jnp.float32),
      mesh=mesh,
      scratch_types=[
          pltpu.VMEM((b_per_w,), jnp.float32),
          pltpu.VMEM((b_per_w,), jnp.float32),
          pltpu.VMEM((b_per_w,), jnp.int32),
          pltpu.VMEM((_NBUF, _CHUNK, _EMBED), jnp.float32),
          pltpu.VMEM((_NBUF, _CHUNK, _EMBED), jnp.float32),
          pltpu.SemaphoreType.DMA((_NBUF,)),
          pltpu.SemaphoreType.DMA((_NBUF,)),
          pltpu.SemaphoreType.DMA((_NBUF,)),
      ],
  )(fx, fy, d, p)
  return out.reshape(shape)
